# baseline (speedup 1.0000x reference)
import jax
import jax.numpy as jnp
from jax import lax
from jax.experimental import pallas as pl
from jax.experimental.pallas import tpu as pltpu
from jax.experimental.pallas import tpu_sc as plsc

_B = 512
_BS = 19
_P = _BS * _BS
_MAXM = _P
_EMPTY = -1.0
_NW = 32
_GPW = _B // _NW
_LANES = 16
_PPAD = 368
_NCHUNK = _PPAD // _LANES


def _sc_body(hist_in, brows_h, bpad_h, roots_h, rows_h, cols_h, mv_h, ply_h,
             cg_h,
             hist_out, board_out,
             brow_v, bpad_v, roots_v, rows_v, cols_v, mv_v, ply_v, cg_v,
             pos_s, play_s, plyf_s,
             sem_big, sem_s):
    wid = lax.axis_index("s") * 2 + lax.axis_index("c")
    base = wid * _GPW
    rbase = base * _MAXM

    big = pltpu.async_copy(hist_in.at[pl.ds(rbase, _GPW * _MAXM)],
                           hist_out.at[pl.ds(rbase, _GPW * _MAXM)], sem_big)

    pltpu.sync_copy(rows_h.at[pl.ds(base, _GPW)], rows_v)
    pltpu.sync_copy(cols_h.at[pl.ds(base, _GPW)], cols_v)
    pltpu.sync_copy(mv_h.at[pl.ds(base, _GPW)], mv_v)
    pltpu.sync_copy(ply_h.at[pl.ds(base, _GPW)], ply_v)
    pltpu.sync_copy(brows_h.at[pl.ds(base, _GPW)], brow_v)
    pltpu.sync_copy(bpad_h.at[pl.ds(base, _GPW)], bpad_v)
    pltpu.sync_copy(roots_h.at[pl.ds(base, _GPW)], roots_v)
    pltpu.sync_copy(cg_h.at[pl.ds(base * _P * 4, _GPW * _P * 4)], cg_v)

    iota = lax.iota(jnp.int32, _LANES)
    rv = rows_v[...]
    cv = cols_v[...]
    mvv = mv_v[...]
    rc = jnp.clip(rv, 0, _BS - 1)
    cc = jnp.clip(cv, 0, _BS - 1)
    posv = rc * _BS + cc
    playv = jnp.where((rv >= 0) & (cv >= 0), jnp.int32(1), jnp.int32(0))
    plyfv = ply_v[...].astype(jnp.float32)

    pos_s[...] = posv
    play_s[...] = playv
    plyf_s[...] = plyfv

    def game_body(i, carry):
        isplat = jnp.full((_LANES,), i, jnp.int32)
        poss = plsc.load_gather(pos_s, [isplat])
        plays = plsc.load_gather(play_s, [isplat]) != 0
        plysf = plsc.load_gather(plyf_s, [isplat])
        cgbase = jnp.full((_LANES,), i * (_P * 4), jnp.int32) + poss * 4
        g0 = plsc.load_gather(cg_v, [cgbase])
        g1 = plsc.load_gather(cg_v, [cgbase + 1])
        g2 = plsc.load_gather(cg_v, [cgbase + 2])
        g3 = plsc.load_gather(cg_v, [cgbase + 3])

        def chunk_body(j, c2):
            off = j * _LANES
            lanes = off + iota
            bvals = bpad_v[i, pl.ds(off, _LANES)]
            rvals = roots_v[i, pl.ds(off, _LANES)]
            v = jnp.where(plays & (lanes == poss), plysf, bvals)
            cap = (((rvals == g0) & (g0 >= 0)) | ((rvals == g1) & (g1 >= 0))
                   | ((rvals == g2) & (g2 >= 0)) | ((rvals == g3) & (g3 >= 0)))
            v = jnp.where(plays & cap, jnp.float32(_EMPTY), v)
            bpad_v[i, pl.ds(off, _LANES)] = v
            return c2

        return lax.fori_loop(0, _NCHUNK, chunk_body, carry)

    lax.fori_loop(0, _GPW, game_body, jnp.int32(0))

    big.wait()

    def scat_body(i, carry):
        mvs = plsc.load_gather(mv_v, [jnp.full((_LANES,), i, jnp.int32)])
        row = (base + i) * _MAXM + mvs[0]
        pltpu.async_copy(brow_v.at[i], hist_out.at[row], sem_s).wait()
        return carry

    lax.fori_loop(0, _GPW, scat_body, jnp.int32(0))

    pltpu.sync_copy(bpad_v, board_out.at[pl.ds(base, _GPW)])


_mesh = plsc.VectorSubcoreMesh(core_axis_name="c", subcore_axis_name="s")

_sc_step = pl.kernel(
    _sc_body,
    out_type=(
        jax.ShapeDtypeStruct((_B * _MAXM, _P), jnp.float32),
        jax.ShapeDtypeStruct((_B, _PPAD), jnp.float32),
    ),
    mesh=_mesh,
    scratch_types=[
        pltpu.VMEM((_GPW, _P), jnp.float32),
        pltpu.VMEM((_GPW, _PPAD), jnp.float32),
        pltpu.VMEM((_GPW, _PPAD), jnp.int32),
        pltpu.VMEM((_GPW,), jnp.int32),
        pltpu.VMEM((_GPW,), jnp.int32),
        pltpu.VMEM((_GPW,), jnp.int32),
        pltpu.VMEM((_GPW,), jnp.int32),
        pltpu.VMEM((_GPW * _P * 4,), jnp.int32),
        pltpu.VMEM((_GPW,), jnp.int32),
        pltpu.VMEM((_GPW,), jnp.int32),
        pltpu.VMEM((_GPW,), jnp.float32),
        pltpu.SemaphoreType.DMA,
        pltpu.SemaphoreType.DMA,
    ],
    compiler_params=pltpu.CompilerParams(needs_layout_passes=False),
    name="go_step_sc",
)


def kernel(board, board_history, positions, current_player, pass_count,
           move_count, roots, capture_groups):
    del pass_count
    board_flat = board.reshape(_B, _P)
    board_pad = jnp.pad(board_flat, ((0, 0), (0, _PPAD - _P)))
    roots_pad = jnp.pad(roots, ((0, 0), (0, _PPAD - _P)), constant_values=-1)
    rows = positions[:, 0]
    cols = positions[:, 1]
    cg1d = capture_groups.reshape(_B * _P * 4)
    hist2d = board_history.reshape(_B * _MAXM, _P)
    hist_out, board_out = _sc_step(hist2d, board_flat, board_pad, roots_pad,
                                   rows, cols, move_count, current_player,
                                   cg1d)
    new_board = board_out[:, :_P].reshape(_B, _BS, _BS)
    return new_board, hist_out.reshape(_B, _MAXM, _P)


# speedup vs baseline: 11.8224x; 11.8224x over previous
import jax
import jax.numpy as jnp
from jax import lax
from jax.experimental import pallas as pl
from jax.experimental.pallas import tpu as pltpu
from jax.experimental.pallas import tpu_sc as plsc

_B = 512
_BS = 19
_P = _BS * _BS
_MAXM = _P
_EMPTY = -1.0
_NW = 32
_GPW = _B // _NW
_LANES = 16
_PPAD = 368
_NCHUNK = _PPAD // _LANES
_GB = 8


def _tc_hist_body(mv_ref, board_ref, hist_ref, out_ref):
    mv = mv_ref[...]
    rows = lax.broadcasted_iota(jnp.int32, (_GB, _MAXM, 1), 1)
    sel = rows == mv[:, :, None]
    out_ref[...] = jnp.where(sel, board_ref[...][:, None, :], hist_ref[...])


_tc_hist = pl.pallas_call(
    _tc_hist_body,
    grid=(_B // _GB,),
    in_specs=[
        pl.BlockSpec((_GB, 1), lambda i: (i, 0)),
        pl.BlockSpec((_GB, _P), lambda i: (i, 0)),
        pl.BlockSpec((_GB, _MAXM, _P), lambda i: (i, 0, 0)),
    ],
    out_specs=pl.BlockSpec((_GB, _MAXM, _P), lambda i: (i, 0, 0)),
    out_shape=jax.ShapeDtypeStruct((_B, _MAXM, _P), jnp.float32),
    compiler_params=pltpu.CompilerParams(
        dimension_semantics=("arbitrary",),
    ),
    name="go_hist_tc",
)


def _sc_board_body(bpad_h, roots_h, rows_h, cols_h, ply_h, cg_h,
                   board_out,
                   bpad_v, roots_v, rows_v, cols_v, ply_v, cg_v,
                   pos_s, play_s, plyf_s):
    wid = lax.axis_index("s") * 2 + lax.axis_index("c")
    base = wid * _GPW

    pltpu.sync_copy(rows_h.at[pl.ds(base, _GPW)], rows_v)
    pltpu.sync_copy(cols_h.at[pl.ds(base, _GPW)], cols_v)
    pltpu.sync_copy(ply_h.at[pl.ds(base, _GPW)], ply_v)
    pltpu.sync_copy(bpad_h.at[pl.ds(base, _GPW)], bpad_v)
    pltpu.sync_copy(roots_h.at[pl.ds(base, _GPW)], roots_v)
    pltpu.sync_copy(cg_h.at[pl.ds(base * _P * 4, _GPW * _P * 4)], cg_v)

    iota = lax.iota(jnp.int32, _LANES)
    rv = rows_v[...]
    cv = cols_v[...]
    rc = jnp.clip(rv, 0, _BS - 1)
    cc = jnp.clip(cv, 0, _BS - 1)
    posv = rc * _BS + cc
    playv = jnp.where((rv >= 0) & (cv >= 0), jnp.int32(1), jnp.int32(0))
    plyfv = ply_v[...].astype(jnp.float32)

    pos_s[...] = posv
    play_s[...] = playv
    plyf_s[...] = plyfv

    def game_body(i, carry):
        isplat = jnp.full((_LANES,), i, jnp.int32)
        poss = plsc.load_gather(pos_s, [isplat])
        plays = plsc.load_gather(play_s, [isplat]) != 0
        plysf = plsc.load_gather(plyf_s, [isplat])
        cgbase = jnp.full((_LANES,), i * (_P * 4), jnp.int32) + poss * 4
        g0 = plsc.load_gather(cg_v, [cgbase])
        g1 = plsc.load_gather(cg_v, [cgbase + 1])
        g2 = plsc.load_gather(cg_v, [cgbase + 2])
        g3 = plsc.load_gather(cg_v, [cgbase + 3])

        def chunk_body(j, c2):
            off = j * _LANES
            lanes = off + iota
            bvals = bpad_v[i, pl.ds(off, _LANES)]
            rvals = roots_v[i, pl.ds(off, _LANES)]
            v = jnp.where(plays & (lanes == poss), plysf, bvals)
            cap = (((rvals == g0) & (g0 >= 0)) | ((rvals == g1) & (g1 >= 0))
                   | ((rvals == g2) & (g2 >= 0)) | ((rvals == g3) & (g3 >= 0)))
            v = jnp.where(plays & cap, jnp.float32(_EMPTY), v)
            bpad_v[i, pl.ds(off, _LANES)] = v
            return c2

        return lax.fori_loop(0, _NCHUNK, chunk_body, carry)

    lax.fori_loop(0, _GPW, game_body, jnp.int32(0))

    pltpu.sync_copy(bpad_v, board_out.at[pl.ds(base, _GPW)])


_mesh = plsc.VectorSubcoreMesh(core_axis_name="c", subcore_axis_name="s")

_sc_board = pl.kernel(
    _sc_board_body,
    out_type=jax.ShapeDtypeStruct((_B, _PPAD), jnp.float32),
    mesh=_mesh,
    scratch_types=[
        pltpu.VMEM((_GPW, _PPAD), jnp.float32),
        pltpu.VMEM((_GPW, _PPAD), jnp.int32),
        pltpu.VMEM((_GPW,), jnp.int32),
        pltpu.VMEM((_GPW,), jnp.int32),
        pltpu.VMEM((_GPW,), jnp.int32),
        pltpu.VMEM((_GPW * _P * 4,), jnp.int32),
        pltpu.VMEM((_GPW,), jnp.int32),
        pltpu.VMEM((_GPW,), jnp.int32),
        pltpu.VMEM((_GPW,), jnp.float32),
    ],
    compiler_params=pltpu.CompilerParams(needs_layout_passes=False),
    name="go_board_sc",
)


def kernel(board, board_history, positions, current_player, pass_count,
           move_count, roots, capture_groups):
    del pass_count
    board_flat = board.reshape(_B, _P)
    board_pad = jnp.pad(board_flat, ((0, 0), (0, _PPAD - _P)))
    roots_pad = jnp.pad(roots, ((0, 0), (0, _PPAD - _P)), constant_values=-1)
    rows = positions[:, 0]
    cols = positions[:, 1]
    cg1d = capture_groups.reshape(_B * _P * 4)
    hist_out = _tc_hist(move_count.reshape(_B, 1), board_flat, board_history)
    board_out = _sc_board(board_pad, roots_pad, rows, cols, current_player,
                          cg1d)
    new_board = board_out[:, :_P].reshape(_B, _BS, _BS)
    return new_board, hist_out
